# baseline (speedup 1.0000x reference)
import jax
import jax.numpy as jnp
from jax import lax
from jax.experimental import pallas as pl
from jax.experimental.pallas import tpu as pltpu
from jax.experimental.pallas import tpu_sc as plsc

D = 64
B = 4096 * 200
NC, NS = 2, 16
NW = NC * NS
ROWS_PER_GATHER = 128
B_PER_W = B // NW
SLABS_PER_W = B_PER_W // ROWS_PER_GATHER
GATHERS_PER_CHUNK = 4
CHUNK = ROWS_PER_GATHER * GATHERS_PER_CHUNK
N_CHUNKS = B_PER_W // CHUNK


def _emb_body(table_hbm, tok_hbm, out_hbm, idx_v, buf0, buf1, gsem0, gsem1,
              osem0, osem1):
    wid = lax.axis_index("s") * NC + lax.axis_index("c")
    base = wid * B_PER_W
    pltpu.sync_copy(tok_hbm.at[pl.ds(wid * SLABS_PER_W, SLABS_PER_W)], idx_v)

    def fire(c, buf, gsem):
        for j in range(GATHERS_PER_CHUNK):
            pltpu.async_copy(
                table_hbm.at[idx_v.at[c * GATHERS_PER_CHUNK + j]],
                buf.at[pl.ds(j * ROWS_PER_GATHER, ROWS_PER_GATHER)],
                gsem,
            )

    def wait_gathers(c, buf, gsem):
        for j in range(GATHERS_PER_CHUNK):
            pltpu.make_async_copy(
                table_hbm.at[idx_v.at[c * GATHERS_PER_CHUNK + j]],
                buf.at[pl.ds(j * ROWS_PER_GATHER, ROWS_PER_GATHER)],
                gsem,
            ).wait()

    def out_slice(c):
        return out_hbm.at[pl.ds(base + c * CHUNK, CHUNK)]

    def store(c, buf, osem):
        pltpu.async_copy(buf, out_slice(c), osem)

    def wait_store(c, buf, osem):
        pltpu.make_async_copy(buf, out_slice(c), osem).wait()

    fire(0, buf0, gsem0)

    def body(i2, carry):
        i = i2 * 2

        @pl.when(i2 > 0)
        def _():
            wait_store(i - 1, buf1, osem1)

        fire(i + 1, buf1, gsem1)
        wait_gathers(i, buf0, gsem0)
        store(i, buf0, osem0)

        wait_store(i, buf0, osem0)

        @pl.when(i + 2 < N_CHUNKS)
        def _():
            fire(i + 2, buf0, gsem0)

        wait_gathers(i + 1, buf1, gsem1)
        store(i + 1, buf1, osem1)
        return carry

    lax.fori_loop(0, N_CHUNKS // 2, body, 0)
    wait_store(N_CHUNKS - 1, buf1, osem1)


def kernel(table, tokens):
    tok2d = tokens.reshape(B // ROWS_PER_GATHER, ROWS_PER_GATHER)
    mesh = plsc.VectorSubcoreMesh(core_axis_name="c", subcore_axis_name="s")
    out = pl.kernel(
        _emb_body,
        mesh=mesh,
        compiler_params=pltpu.CompilerParams(use_tc_tiling_on_sc=False),
        out_type=jax.ShapeDtypeStruct((B, D), jnp.float32),
        scratch_types=[
            pltpu.VMEM((SLABS_PER_W, ROWS_PER_GATHER), jnp.int32),
            pltpu.VMEM((CHUNK, D), jnp.float32),
            pltpu.VMEM((CHUNK, D), jnp.float32),
            pltpu.SemaphoreType.DMA,
            pltpu.SemaphoreType.DMA,
            pltpu.SemaphoreType.DMA,
            pltpu.SemaphoreType.DMA,
        ],
    )(table, tok2d)
    return out.reshape(tokens.shape[0], tokens.shape[1], D)
